# Initial kernel scaffold; baseline (speedup 1.0000x reference)
import jax
import jax.numpy as jnp
from jax.experimental import pallas as pl


def kernel(utts, utts_t, meanings_t):
    raise NotImplementedError("write your pallas kernel here")


# speedup vs baseline: 2.6300x; 2.6300x over previous
import functools

import jax
import jax.numpy as jnp
from jax import lax
from jax.experimental import pallas as pl
from jax.experimental.pallas import tpu as pltpu
from jax.experimental.pallas import tpu_sc as plsc

VOCAB = 64
SEQ = 8
NQ = 1024
NSUP = 100000
FEAT = SEQ * VOCAB
CHUNK = 2000
NTYPES = 5
NMEAN = 10
OUTW = NTYPES * NMEAN
IDX_SCALE = 131072.0

def _i32(x):
    return jnp.asarray(x, jnp.int32)


def _argmax_body(utts_ref, sup_ref, out_ref, q1h_ref, best_ref):
    i = pl.program_id(0)
    nsteps = pl.num_programs(0)

    @pl.when(i == 0)
    def _init():
        best_ref[...] = jnp.full((NQ, 1), -1.0, jnp.float32)
        for p in range(SEQ):
            tok = utts_ref[:, p : p + 1]
            v = lax.broadcasted_iota(jnp.int32, (NQ, VOCAB), 1)
            q1h_ref[:, p * VOCAB : (p + 1) * VOCAB] = (tok == v).astype(
                jnp.bfloat16
            )

    chunk = sup_ref[...].astype(jnp.bfloat16)
    matches = lax.dot_general(
        q1h_ref[...],
        chunk,
        (((1,), (1,)), ((), ())),
        preferred_element_type=jnp.float32,
    )
    col = lax.broadcasted_iota(jnp.int32, (1, CHUNK), 1).astype(jnp.float32)
    base = (i * CHUNK).astype(jnp.float32)
    key = matches * IDX_SCALE + ((IDX_SCALE - 1.0) - (base + col))
    step_best = jnp.max(key, axis=1, keepdims=True)
    best_ref[...] = jnp.maximum(best_ref[...], step_best)

    @pl.when(i == nsteps - 1)
    def _fin():
        b = best_ref[...]
        m = jnp.floor(b / IDX_SCALE)
        rem = b - m * IDX_SCALE
        out_ref[...] = ((IDX_SCALE - 1.0) - rem).astype(jnp.int32)


def _best_index(utts_q, utts_t, interpret=False):
    return pl.pallas_call(
        _argmax_body,
        grid=(NSUP // CHUNK,),
        in_specs=[
            pl.BlockSpec((NQ, SEQ), lambda i: (_i32(0), _i32(0))),
            pl.BlockSpec((CHUNK, FEAT), lambda i: (_i32(i), _i32(0))),
        ],
        out_specs=pl.BlockSpec((NQ, 1), lambda i: (_i32(0), _i32(0))),
        out_shape=jax.ShapeDtypeStruct((NQ, 1), jnp.int32),
        scratch_shapes=[
            pltpu.VMEM((NQ, FEAT), jnp.bfloat16),
            pltpu.VMEM((NQ, 1), jnp.float32),
        ],
        interpret=interpret,
    )(utts_q, utts_t)


_SC_WORKERS = 32
_BW = NQ // _SC_WORKERS
_TABW = 16


def _gather_onehot_body(idx_hbm, tab_hbm, out_hbm, idx_v, rows_v, out_v, sem):
    wid = lax.axis_index("s") * 2 + lax.axis_index("c")
    base = wid * _BW
    pltpu.sync_copy(idx_hbm.at[pl.ds(base, _BW)], idx_v)
    cp = pltpu.async_copy(tab_hbm.at[idx_v], rows_v, sem)

    zeros16 = jnp.zeros((16,), jnp.float32)
    for j in range(_BW * OUTW // 16):
        out_v[pl.ds(j * 16, 16)] = zeros16
    cp.wait()

    ones16 = jnp.ones((16,), jnp.float32)
    lanes = lax.iota(jnp.int32, 16)
    tmask = lanes < NTYPES
    for j in range(_BW):
        r = rows_v[j]
        oidx = j * OUTW + lanes * NMEAN + r
        plsc.store_scatter(out_v, [oidx], ones16, mask=tmask)

    pltpu.sync_copy(out_v, out_hbm.at[pl.ds(base * OUTW, _BW * OUTW)])


def _make_gather_onehot():
    mesh = plsc.VectorSubcoreMesh(core_axis_name="c", subcore_axis_name="s")
    return pl.kernel(
        _gather_onehot_body,
        mesh=mesh,
        compiler_params=pltpu.CompilerParams(
            needs_layout_passes=False, use_tc_tiling_on_sc=False
        ),
        out_type=jax.ShapeDtypeStruct((NQ * OUTW,), jnp.float32),
        scratch_types=[
            pltpu.VMEM((_BW,), jnp.int32),
            pltpu.VMEM((_BW, _TABW), jnp.int32),
            pltpu.VMEM((_BW * OUTW,), jnp.float32),
            pltpu.SemaphoreType.DMA,
        ],
    )


def kernel(utts, utts_t, meanings_t):
    utts_q = jnp.transpose(utts).astype(jnp.int32)
    tab = jnp.pad(
        meanings_t.astype(jnp.int32), ((0, 0), (0, _TABW - NTYPES))
    )
    best = _best_index(utts_q, utts_t)
    idx = best.reshape(NQ)
    flat = _make_gather_onehot()(idx, tab)
    return flat.reshape(NQ, NTYPES, NMEAN)


# speedup vs baseline: 2.6305x; 1.0002x over previous
import functools

import jax
import jax.numpy as jnp
from jax import lax
from jax.experimental import pallas as pl
from jax.experimental.pallas import tpu as pltpu
from jax.experimental.pallas import tpu_sc as plsc

VOCAB = 64
SEQ = 8
NQ = 1024
NSUP = 100000
FEAT = SEQ * VOCAB
CHUNK = 2000
NTYPES = 5
NMEAN = 10
OUTW = NTYPES * NMEAN
IDX_SCALE = 131072.0

def _i32(x):
    return jnp.asarray(x, jnp.int32)


def _argmax_body(utts_ref, sup_ref, out_ref, q1h_ref, best_ref):
    i = pl.program_id(0)
    nsteps = pl.num_programs(0)

    @pl.when(i == 0)
    def _init():
        best_ref[...] = jnp.full((NQ, 1), -1.0, jnp.float32)
        for p in range(SEQ):
            tok = utts_ref[:, p : p + 1]
            v = lax.broadcasted_iota(jnp.int32, (NQ, VOCAB), 1)
            q1h_ref[:, p * VOCAB : (p + 1) * VOCAB] = (tok == v).astype(
                jnp.bfloat16
            )

    chunk = sup_ref[...].astype(jnp.bfloat16)
    matches = lax.dot_general(
        q1h_ref[...],
        chunk,
        (((1,), (1,)), ((), ())),
        preferred_element_type=jnp.float32,
    )
    col = lax.broadcasted_iota(jnp.int32, (1, CHUNK), 1).astype(jnp.float32)
    base = (i * CHUNK).astype(jnp.float32)
    key = matches * IDX_SCALE + ((IDX_SCALE - 1.0) - (base + col))
    step_best = jnp.max(key, axis=1, keepdims=True)
    best_ref[...] = jnp.maximum(best_ref[...], step_best)

    @pl.when(i == nsteps - 1)
    def _fin():
        b = best_ref[...]
        m = jnp.floor(b / IDX_SCALE)
        rem = b - m * IDX_SCALE
        out_ref[...] = ((IDX_SCALE - 1.0) - rem).astype(jnp.int32)


def _best_index(utts_q, utts_t, interpret=False):
    return pl.pallas_call(
        _argmax_body,
        grid=(NSUP // CHUNK,),
        in_specs=[
            pl.BlockSpec((NQ, SEQ), lambda i: (_i32(0), _i32(0))),
            pl.BlockSpec((CHUNK, FEAT), lambda i: (_i32(i), _i32(0))),
        ],
        out_specs=pl.BlockSpec((NQ, 1), lambda i: (_i32(0), _i32(0))),
        out_shape=jax.ShapeDtypeStruct((NQ, 1), jnp.int32),
        scratch_shapes=[
            pltpu.VMEM((NQ, FEAT), jnp.bfloat16),
            pltpu.VMEM((NQ, 1), jnp.float32),
        ],
        interpret=interpret,
    )(utts_q, utts_t)


_SC_WORKERS = 32
_BW = NQ // _SC_WORKERS
_TABW = 16


def _gather_onehot_body(idx_hbm, tab_hbm, out_hbm, idx_v, rows_v, out_v, sem):
    wid = lax.axis_index("s") * 2 + lax.axis_index("c")
    base = wid * _BW
    pltpu.sync_copy(idx_hbm.at[pl.ds(base, _BW)], idx_v)
    cp = pltpu.async_copy(tab_hbm.at[idx_v], rows_v, sem)

    zeros16 = jnp.zeros((16,), jnp.float32)
    for j in range(_BW * OUTW // 16):
        out_v[pl.ds(j * 16, 16)] = zeros16
    cp.wait()

    ones16 = jnp.ones((16,), jnp.float32)
    lanes = lax.iota(jnp.int32, 16)
    tmask = lanes < NTYPES
    for j in range(_BW):
        r = rows_v[j]
        oidx = j * OUTW + lanes * NMEAN + r
        plsc.store_scatter(out_v, [oidx], ones16, mask=tmask)

    pltpu.sync_copy(out_v, out_hbm.at[pl.ds(base * OUTW, _BW * OUTW)])


def _make_gather_onehot():
    mesh = plsc.VectorSubcoreMesh(core_axis_name="c", subcore_axis_name="s")
    return pl.kernel(
        _gather_onehot_body,
        mesh=mesh,
        compiler_params=pltpu.CompilerParams(
            needs_layout_passes=False,
            use_tc_tiling_on_sc=False,
            skip_device_barrier=True,
        ),
        out_type=jax.ShapeDtypeStruct((NQ * OUTW,), jnp.float32),
        scratch_types=[
            pltpu.VMEM((_BW,), jnp.int32),
            pltpu.VMEM((_BW, _TABW), jnp.int32),
            pltpu.VMEM((_BW * OUTW,), jnp.float32),
            pltpu.SemaphoreType.DMA,
        ],
    )


def kernel(utts, utts_t, meanings_t):
    utts_q = jnp.transpose(utts).astype(jnp.int32)
    tab = jnp.pad(
        meanings_t.astype(jnp.int32), ((0, 0), (0, _TABW - NTYPES))
    )
    best = _best_index(utts_q, utts_t)
    idx = best.reshape(NQ)
    flat = _make_gather_onehot()(idx, tab)
    return flat.reshape(NQ, NTYPES, NMEAN)


# speedup vs baseline: 2.6436x; 1.0050x over previous
import functools

import jax
import jax.numpy as jnp
from jax import lax
from jax.experimental import pallas as pl
from jax.experimental.pallas import tpu as pltpu
from jax.experimental.pallas import tpu_sc as plsc

VOCAB = 64
SEQ = 8
NQ = 1024
NSUP = 100000
FEAT = SEQ * VOCAB
CHUNK = 2000
NTYPES = 5
NMEAN = 10
OUTW = NTYPES * NMEAN
IDX_SCALE = 131072.0

def _i32(x):
    return jnp.asarray(x, jnp.int32)


def _argmax_body(utts_ref, sup_ref, out_ref, q1h_ref, best_ref):
    i = pl.program_id(0)
    nsteps = pl.num_programs(0)

    @pl.when(i == 0)
    def _init():
        best_ref[...] = jnp.full((NQ, 1), -1.0, jnp.float32)
        for p in range(SEQ):
            tok = utts_ref[:, p : p + 1]
            v = lax.broadcasted_iota(jnp.int32, (NQ, VOCAB), 1)
            q1h_ref[:, p * VOCAB : (p + 1) * VOCAB] = (tok == v).astype(
                jnp.bfloat16
            )

    chunk = sup_ref[...].astype(jnp.bfloat16)
    matches = lax.dot_general(
        q1h_ref[...],
        chunk,
        (((1,), (1,)), ((), ())),
        preferred_element_type=jnp.float32,
    )
    col = lax.broadcasted_iota(jnp.int32, (1, CHUNK), 1).astype(jnp.float32)
    base = (i * CHUNK).astype(jnp.float32)
    key = matches * IDX_SCALE + ((IDX_SCALE - 1.0) - (base + col))
    step_best = jnp.max(key, axis=1, keepdims=True)
    best_ref[...] = jnp.maximum(best_ref[...], step_best)

    @pl.when(i == nsteps - 1)
    def _fin():
        b = best_ref[...]
        m = jnp.floor(b / IDX_SCALE)
        rem = b - m * IDX_SCALE
        out_ref[...] = ((IDX_SCALE - 1.0) - rem).astype(jnp.int32)


def _best_index(utts_q, utts_t, interpret=False):
    return pl.pallas_call(
        _argmax_body,
        grid=(NSUP // CHUNK,),
        in_specs=[
            pl.BlockSpec((NQ, SEQ), lambda i: (_i32(0), _i32(0))),
            pl.BlockSpec((CHUNK, FEAT), lambda i: (_i32(i), _i32(0))),
        ],
        out_specs=pl.BlockSpec((NQ, 1), lambda i: (_i32(0), _i32(0))),
        out_shape=jax.ShapeDtypeStruct((NQ, 1), jnp.int32),
        scratch_shapes=[
            pltpu.VMEM((NQ, FEAT), jnp.bfloat16),
            pltpu.VMEM((NQ, 1), jnp.float32),
        ],
        interpret=interpret,
    )(utts_q, utts_t)


_SC_CORES = 1
_SC_WORKERS = 16 * _SC_CORES
_BW = NQ // _SC_WORKERS
_TABW = 16


def _gather_onehot_body(idx_hbm, tab_hbm, out_hbm, idx_v, rows_v, out_v, sem):
    wid = lax.axis_index("s") * _SC_CORES + lax.axis_index("c")
    base = wid * _BW
    pltpu.sync_copy(idx_hbm.at[pl.ds(base, _BW)], idx_v)
    cp = pltpu.async_copy(tab_hbm.at[idx_v], rows_v, sem)

    zeros16 = jnp.zeros((16,), jnp.float32)
    for j in range(_BW * OUTW // 16):
        out_v[pl.ds(j * 16, 16)] = zeros16
    cp.wait()

    ones16 = jnp.ones((16,), jnp.float32)
    lanes = lax.iota(jnp.int32, 16)
    tmask = lanes < NTYPES
    for j in range(_BW):
        r = rows_v[j]
        oidx = j * OUTW + lanes * NMEAN + r
        plsc.store_scatter(out_v, [oidx], ones16, mask=tmask)

    pltpu.sync_copy(out_v, out_hbm.at[pl.ds(base * OUTW, _BW * OUTW)])


def _make_gather_onehot():
    mesh = plsc.VectorSubcoreMesh(
        core_axis_name="c", subcore_axis_name="s", num_cores=_SC_CORES
    )
    return pl.kernel(
        _gather_onehot_body,
        mesh=mesh,
        compiler_params=pltpu.CompilerParams(
            needs_layout_passes=False,
            use_tc_tiling_on_sc=False,
            skip_device_barrier=True,
        ),
        out_type=jax.ShapeDtypeStruct((NQ * OUTW,), jnp.float32),
        scratch_types=[
            pltpu.VMEM((_BW,), jnp.int32),
            pltpu.VMEM((_BW, _TABW), jnp.int32),
            pltpu.VMEM((_BW * OUTW,), jnp.float32),
            pltpu.SemaphoreType.DMA,
        ],
    )


def kernel(utts, utts_t, meanings_t):
    utts_q = jnp.transpose(utts).astype(jnp.int32)
    tab = jnp.pad(
        meanings_t.astype(jnp.int32), ((0, 0), (0, _TABW - NTYPES))
    )
    best = _best_index(utts_q, utts_t)
    idx = best.reshape(NQ)
    flat = _make_gather_onehot()(idx, tab)
    return flat.reshape(NQ, NTYPES, NMEAN)


# speedup vs baseline: 8.8564x; 3.3502x over previous
import functools

import jax
import jax.numpy as jnp
from jax import lax
from jax.experimental import pallas as pl
from jax.experimental.pallas import tpu as pltpu
from jax.experimental.pallas import tpu_sc as plsc

VOCAB = 64
SEQ = 8
NQ = 1024
NSUP = 100000
FEAT = SEQ * VOCAB
CHUNK = 2000
NTYPES = 5
NMEAN = 10
OUTW = NTYPES * NMEAN
IDX_SCALE = 131072.0

def _i32(x):
    return jnp.asarray(x, jnp.int32)


def _argmax_body(utts_ref, sup_ref, out_ref, q1h_ref, best_ref):
    i = pl.program_id(0)
    nsteps = pl.num_programs(0)

    @pl.when(i == 0)
    def _init():
        best_ref[...] = jnp.full((NQ, 1), -1.0, jnp.float32)
        for p in range(SEQ):
            tok = utts_ref[:, p : p + 1]
            v = lax.broadcasted_iota(jnp.int32, (NQ, VOCAB), 1)
            q1h_ref[:, p * VOCAB : (p + 1) * VOCAB] = (tok == v).astype(
                jnp.bfloat16
            )

    chunk = sup_ref[...].astype(jnp.bfloat16)
    matches = lax.dot_general(
        q1h_ref[...],
        chunk,
        (((1,), (1,)), ((), ())),
        preferred_element_type=jnp.float32,
    )
    col = lax.broadcasted_iota(jnp.int32, (1, CHUNK), 1).astype(jnp.float32)
    base = (i * CHUNK).astype(jnp.float32)
    key = matches * IDX_SCALE + ((IDX_SCALE - 1.0) - (base + col))
    step_best = jnp.max(key, axis=1, keepdims=True)
    best_ref[...] = jnp.maximum(best_ref[...], step_best)

    @pl.when(i == nsteps - 1)
    def _fin():
        b = best_ref[...]
        m = jnp.floor(b / IDX_SCALE)
        rem = b - m * IDX_SCALE
        out_ref[...] = ((IDX_SCALE - 1.0) - rem).astype(jnp.int32)


def _best_index(utts_q, utts_t, interpret=False):
    return pl.pallas_call(
        _argmax_body,
        grid=(NSUP // CHUNK,),
        in_specs=[
            pl.BlockSpec((NQ, SEQ), lambda i: (_i32(0), _i32(0))),
            pl.BlockSpec((CHUNK, FEAT), lambda i: (_i32(i), _i32(0))),
        ],
        out_specs=pl.BlockSpec((NQ, 1), lambda i: (_i32(0), _i32(0))),
        out_shape=jax.ShapeDtypeStruct((NQ, 1), jnp.int32),
        scratch_shapes=[
            pltpu.VMEM((NQ, FEAT), jnp.bfloat16),
            pltpu.VMEM((NQ, 1), jnp.float32),
        ],
        interpret=interpret,
    )(utts_q, utts_t)


_SC_CORES = 2
_SC_WORKERS = 16 * _SC_CORES
_BW = NQ // _SC_WORKERS
_GW = _BW // 16 * NTYPES * 16


def _gather_onehot_body(
    idx_hbm, tabw_hbm, out_hbm, idx_v, w0_v, w1_v, r0_v, r1_v, out_v, sem
):
    wid = lax.axis_index("s") * _SC_CORES + lax.axis_index("c")
    base = wid * _BW
    pltpu.sync_copy(idx_hbm.at[pl.ds(base, _BW)], idx_v)

    lanes = lax.iota(jnp.int32, 16)
    for g in range(2):
        q16 = idx_v[pl.ds(g * 16, 16)]
        wv = w0_v if g == 0 else w1_v
        for k in range(NTYPES):
            plsc.store_scatter(wv, [lanes * NTYPES + k], q16 + k * NSUP)
    cp0 = pltpu.async_copy(tabw_hbm.at[w0_v], r0_v, sem)
    cp1 = pltpu.async_copy(tabw_hbm.at[w1_v], r1_v, sem)

    zeros16 = jnp.zeros((16,), jnp.float32)
    for j in range(_BW * OUTW // 16):
        out_v[pl.ds(j * 16, 16)] = zeros16
    cp0.wait()
    cp1.wait()

    ones16 = jnp.ones((16,), jnp.float32)
    for g in range(2):
        rv = r0_v if g == 0 else r1_v
        for k in range(NTYPES):
            m16 = plsc.load_gather(rv, [lanes * NTYPES + k])
            oidx = (g * 16 + lanes) * OUTW + k * NMEAN + m16
            plsc.store_scatter(out_v, [oidx], ones16)

    pltpu.sync_copy(out_v, out_hbm.at[pl.ds(base * OUTW, _BW * OUTW)])


def _make_gather_onehot():
    mesh = plsc.VectorSubcoreMesh(
        core_axis_name="c", subcore_axis_name="s", num_cores=_SC_CORES
    )
    return pl.kernel(
        _gather_onehot_body,
        mesh=mesh,
        compiler_params=pltpu.CompilerParams(
            needs_layout_passes=False,
            use_tc_tiling_on_sc=False,
            skip_device_barrier=True,
        ),
        out_type=jax.ShapeDtypeStruct((NQ * OUTW,), jnp.float32),
        scratch_types=[
            pltpu.VMEM((_BW,), jnp.int32),
            pltpu.VMEM((_GW,), jnp.int32),
            pltpu.VMEM((_GW,), jnp.int32),
            pltpu.VMEM((_GW,), jnp.int32),
            pltpu.VMEM((_GW,), jnp.int32),
            pltpu.VMEM((_BW * OUTW,), jnp.float32),
            pltpu.SemaphoreType.DMA,
        ],
    )


def kernel(utts, utts_t, meanings_t):
    utts_q = jnp.transpose(utts).astype(jnp.int32)
    tabw = jnp.transpose(meanings_t, (1, 0)).astype(jnp.int32).reshape(-1)
    best = _best_index(utts_q, utts_t)
    idx = best.reshape(NQ)
    flat = _make_gather_onehot()(idx, tabw)
    return flat.reshape(NQ, NTYPES, NMEAN)


# speedup vs baseline: 9.1377x; 1.0318x over previous
import functools

import jax
import jax.numpy as jnp
from jax import lax
from jax.experimental import pallas as pl
from jax.experimental.pallas import tpu as pltpu
from jax.experimental.pallas import tpu_sc as plsc

VOCAB = 64
SEQ = 8
NQ = 1024
NSUP = 100000
FEAT = SEQ * VOCAB
CHUNK = 4000
NTYPES = 5
NMEAN = 10
OUTW = NTYPES * NMEAN
IDX_SCALE = 131072.0

def _i32(x):
    return jnp.asarray(x, jnp.int32)


def _argmax_body(utts_ref, sup_ref, out_ref, q1h_ref, best_ref):
    i = pl.program_id(0)
    nsteps = pl.num_programs(0)

    @pl.when(i == 0)
    def _init():
        best_ref[...] = jnp.full((NQ, 1), -1.0, jnp.float32)
        for p in range(SEQ):
            tok = utts_ref[:, p : p + 1]
            v = lax.broadcasted_iota(jnp.int32, (NQ, VOCAB), 1)
            q1h_ref[:, p * VOCAB : (p + 1) * VOCAB] = (tok == v).astype(
                jnp.bfloat16
            )

    chunk = sup_ref[...].astype(jnp.bfloat16)
    matches = lax.dot_general(
        q1h_ref[...],
        chunk,
        (((1,), (1,)), ((), ())),
        preferred_element_type=jnp.float32,
    )
    col = lax.broadcasted_iota(jnp.int32, (1, CHUNK), 1).astype(jnp.float32)
    base = (i * CHUNK).astype(jnp.float32)
    key = matches * IDX_SCALE + ((IDX_SCALE - 1.0) - (base + col))
    step_best = jnp.max(key, axis=1, keepdims=True)
    best_ref[...] = jnp.maximum(best_ref[...], step_best)

    @pl.when(i == nsteps - 1)
    def _fin():
        b = best_ref[...]
        m = jnp.floor(b / IDX_SCALE)
        rem = b - m * IDX_SCALE
        out_ref[...] = ((IDX_SCALE - 1.0) - rem).astype(jnp.int32)


def _best_index(utts_q, utts_t, interpret=False):
    return pl.pallas_call(
        _argmax_body,
        grid=(NSUP // CHUNK,),
        in_specs=[
            pl.BlockSpec((NQ, SEQ), lambda i: (_i32(0), _i32(0))),
            pl.BlockSpec((CHUNK, FEAT), lambda i: (_i32(i), _i32(0))),
        ],
        out_specs=pl.BlockSpec((NQ, 1), lambda i: (_i32(0), _i32(0))),
        out_shape=jax.ShapeDtypeStruct((NQ, 1), jnp.int32),
        scratch_shapes=[
            pltpu.VMEM((NQ, FEAT), jnp.bfloat16),
            pltpu.VMEM((NQ, 1), jnp.float32),
        ],
        interpret=interpret,
    )(utts_q, utts_t)


_SC_CORES = 2
_SC_WORKERS = 16 * _SC_CORES
_BW = NQ // _SC_WORKERS
_GW = _BW // 16 * NTYPES * 16


def _gather_onehot_body(
    idx_hbm, tabw_hbm, out_hbm, idx_v, w0_v, w1_v, r0_v, r1_v, out_v, sem
):
    wid = lax.axis_index("s") * _SC_CORES + lax.axis_index("c")
    base = wid * _BW
    pltpu.sync_copy(idx_hbm.at[pl.ds(base, _BW)], idx_v)

    lanes = lax.iota(jnp.int32, 16)
    for g in range(2):
        q16 = idx_v[pl.ds(g * 16, 16)]
        wv = w0_v if g == 0 else w1_v
        for k in range(NTYPES):
            plsc.store_scatter(wv, [lanes * NTYPES + k], q16 + k * NSUP)
    cp0 = pltpu.async_copy(tabw_hbm.at[w0_v], r0_v, sem)
    cp1 = pltpu.async_copy(tabw_hbm.at[w1_v], r1_v, sem)

    zeros16 = jnp.zeros((16,), jnp.float32)
    for j in range(_BW * OUTW // 16):
        out_v[pl.ds(j * 16, 16)] = zeros16
    cp0.wait()
    cp1.wait()

    ones16 = jnp.ones((16,), jnp.float32)
    for g in range(2):
        rv = r0_v if g == 0 else r1_v
        for k in range(NTYPES):
            m16 = plsc.load_gather(rv, [lanes * NTYPES + k])
            oidx = (g * 16 + lanes) * OUTW + k * NMEAN + m16
            plsc.store_scatter(out_v, [oidx], ones16)

    pltpu.sync_copy(out_v, out_hbm.at[pl.ds(base * OUTW, _BW * OUTW)])


def _make_gather_onehot():
    mesh = plsc.VectorSubcoreMesh(
        core_axis_name="c", subcore_axis_name="s", num_cores=_SC_CORES
    )
    return pl.kernel(
        _gather_onehot_body,
        mesh=mesh,
        compiler_params=pltpu.CompilerParams(
            needs_layout_passes=False,
            use_tc_tiling_on_sc=False,
            skip_device_barrier=True,
        ),
        out_type=jax.ShapeDtypeStruct((NQ * OUTW,), jnp.float32),
        scratch_types=[
            pltpu.VMEM((_BW,), jnp.int32),
            pltpu.VMEM((_GW,), jnp.int32),
            pltpu.VMEM((_GW,), jnp.int32),
            pltpu.VMEM((_GW,), jnp.int32),
            pltpu.VMEM((_GW,), jnp.int32),
            pltpu.VMEM((_BW * OUTW,), jnp.float32),
            pltpu.SemaphoreType.DMA,
        ],
    )


def kernel(utts, utts_t, meanings_t):
    utts_q = jnp.transpose(utts).astype(jnp.int32)
    tabw = jnp.transpose(meanings_t, (1, 0)).astype(jnp.int32).reshape(-1)
    best = _best_index(utts_q, utts_t)
    idx = best.reshape(NQ)
    flat = _make_gather_onehot()(idx, tabw)
    return flat.reshape(NQ, NTYPES, NMEAN)


# speedup vs baseline: 9.1803x; 1.0047x over previous
import functools

import jax
import jax.numpy as jnp
from jax import lax
from jax.experimental import pallas as pl
from jax.experimental.pallas import tpu as pltpu
from jax.experimental.pallas import tpu_sc as plsc

VOCAB = 64
SEQ = 8
NQ = 1024
NSUP = 100000
FEAT = SEQ * VOCAB
CHUNK = 5000
NTYPES = 5
NMEAN = 10
OUTW = NTYPES * NMEAN
IDX_SCALE = 131072.0

def _i32(x):
    return jnp.asarray(x, jnp.int32)


def _argmax_body(utts_ref, sup_ref, out_ref, q1h_ref, best_ref):
    i = pl.program_id(0)
    nsteps = pl.num_programs(0)

    @pl.when(i == 0)
    def _init():
        best_ref[...] = jnp.full((NQ, 1), -1.0, jnp.float32)
        for p in range(SEQ):
            tok = utts_ref[:, p : p + 1]
            v = lax.broadcasted_iota(jnp.int32, (NQ, VOCAB), 1)
            q1h_ref[:, p * VOCAB : (p + 1) * VOCAB] = (tok == v).astype(
                jnp.bfloat16
            )

    chunk = sup_ref[...].astype(jnp.bfloat16)
    matches = lax.dot_general(
        q1h_ref[...],
        chunk,
        (((1,), (1,)), ((), ())),
        preferred_element_type=jnp.float32,
    )
    col = lax.broadcasted_iota(jnp.int32, (1, CHUNK), 1).astype(jnp.float32)
    base = (i * CHUNK).astype(jnp.float32)
    key = matches * IDX_SCALE + ((IDX_SCALE - 1.0) - (base + col))
    step_best = jnp.max(key, axis=1, keepdims=True)
    best_ref[...] = jnp.maximum(best_ref[...], step_best)

    @pl.when(i == nsteps - 1)
    def _fin():
        b = best_ref[...]
        m = jnp.floor(b / IDX_SCALE)
        rem = b - m * IDX_SCALE
        out_ref[...] = ((IDX_SCALE - 1.0) - rem).astype(jnp.int32)


def _best_index(utts_q, utts_t, interpret=False):
    return pl.pallas_call(
        _argmax_body,
        grid=(NSUP // CHUNK,),
        in_specs=[
            pl.BlockSpec((NQ, SEQ), lambda i: (_i32(0), _i32(0))),
            pl.BlockSpec((CHUNK, FEAT), lambda i: (_i32(i), _i32(0))),
        ],
        out_specs=pl.BlockSpec((NQ, 1), lambda i: (_i32(0), _i32(0))),
        out_shape=jax.ShapeDtypeStruct((NQ, 1), jnp.int32),
        scratch_shapes=[
            pltpu.VMEM((NQ, FEAT), jnp.bfloat16),
            pltpu.VMEM((NQ, 1), jnp.float32),
        ],
        interpret=interpret,
    )(utts_q, utts_t)


_SC_CORES = 2
_SC_WORKERS = 16 * _SC_CORES
_BW = NQ // _SC_WORKERS
_GW = _BW // 16 * NTYPES * 16


def _gather_onehot_body(
    idx_hbm, tabw_hbm, out_hbm, idx_v, w0_v, w1_v, r0_v, r1_v, out_v, sem
):
    wid = lax.axis_index("s") * _SC_CORES + lax.axis_index("c")
    base = wid * _BW
    pltpu.sync_copy(idx_hbm.at[pl.ds(base, _BW)], idx_v)

    lanes = lax.iota(jnp.int32, 16)
    for g in range(2):
        q16 = idx_v[pl.ds(g * 16, 16)]
        wv = w0_v if g == 0 else w1_v
        for k in range(NTYPES):
            plsc.store_scatter(wv, [lanes * NTYPES + k], q16 + k * NSUP)
    cp0 = pltpu.async_copy(tabw_hbm.at[w0_v], r0_v, sem)
    cp1 = pltpu.async_copy(tabw_hbm.at[w1_v], r1_v, sem)

    zeros16 = jnp.zeros((16,), jnp.float32)
    for j in range(_BW * OUTW // 16):
        out_v[pl.ds(j * 16, 16)] = zeros16
    cp0.wait()
    cp1.wait()

    ones16 = jnp.ones((16,), jnp.float32)
    for g in range(2):
        rv = r0_v if g == 0 else r1_v
        for k in range(NTYPES):
            m16 = plsc.load_gather(rv, [lanes * NTYPES + k])
            oidx = (g * 16 + lanes) * OUTW + k * NMEAN + m16
            plsc.store_scatter(out_v, [oidx], ones16)

    pltpu.sync_copy(out_v, out_hbm.at[pl.ds(base * OUTW, _BW * OUTW)])


def _make_gather_onehot():
    mesh = plsc.VectorSubcoreMesh(
        core_axis_name="c", subcore_axis_name="s", num_cores=_SC_CORES
    )
    return pl.kernel(
        _gather_onehot_body,
        mesh=mesh,
        compiler_params=pltpu.CompilerParams(
            needs_layout_passes=False,
            use_tc_tiling_on_sc=False,
            skip_device_barrier=True,
        ),
        out_type=jax.ShapeDtypeStruct((NQ * OUTW,), jnp.float32),
        scratch_types=[
            pltpu.VMEM((_BW,), jnp.int32),
            pltpu.VMEM((_GW,), jnp.int32),
            pltpu.VMEM((_GW,), jnp.int32),
            pltpu.VMEM((_GW,), jnp.int32),
            pltpu.VMEM((_GW,), jnp.int32),
            pltpu.VMEM((_BW * OUTW,), jnp.float32),
            pltpu.SemaphoreType.DMA,
        ],
    )


def kernel(utts, utts_t, meanings_t):
    utts_q = jnp.transpose(utts).astype(jnp.int32)
    tabw = jnp.transpose(meanings_t, (1, 0)).astype(jnp.int32).reshape(-1)
    best = _best_index(utts_q, utts_t)
    idx = best.reshape(NQ)
    flat = _make_gather_onehot()(idx, tabw)
    return flat.reshape(NQ, NTYPES, NMEAN)


# speedup vs baseline: 9.1865x; 1.0007x over previous
import functools

import jax
import jax.numpy as jnp
from jax import lax
from jax.experimental import pallas as pl
from jax.experimental.pallas import tpu as pltpu
from jax.experimental.pallas import tpu_sc as plsc

VOCAB = 64
SEQ = 8
NQ = 1024
NSUP = 100000
FEAT = SEQ * VOCAB
CHUNK = 5000
NTYPES = 5
NMEAN = 10
OUTW = NTYPES * NMEAN
IDX_SCALE = 131072.0

def _i32(x):
    return jnp.asarray(x, jnp.int32)


def _argmax_body(utts_ref, sup_ref, out_ref, q1h_ref, best_ref):
    i = pl.program_id(0)
    nsteps = pl.num_programs(0)

    @pl.when(i == 0)
    def _init():
        best_ref[...] = jnp.full((NQ, 1), -1.0, jnp.float32)
        for p in range(SEQ):
            tok = utts_ref[:, p : p + 1]
            v = lax.broadcasted_iota(jnp.int32, (NQ, VOCAB), 1)
            q1h_ref[:, p * VOCAB : (p + 1) * VOCAB] = (
                (tok == v).astype(jnp.float32) * IDX_SCALE
            ).astype(jnp.bfloat16)

    chunk = sup_ref[...].astype(jnp.bfloat16)
    matches = lax.dot_general(
        q1h_ref[...],
        chunk,
        (((1,), (1,)), ((), ())),
        preferred_element_type=jnp.float32,
    )
    col = lax.broadcasted_iota(jnp.int32, (1, CHUNK), 1).astype(jnp.float32)
    base = (i * CHUNK).astype(jnp.float32)
    key = matches + ((IDX_SCALE - 1.0) - (base + col))
    step_best = jnp.max(key, axis=1, keepdims=True)
    best_ref[...] = jnp.maximum(best_ref[...], step_best)

    @pl.when(i == nsteps - 1)
    def _fin():
        b = best_ref[...]
        m = jnp.floor(b / IDX_SCALE)
        rem = b - m * IDX_SCALE
        out_ref[...] = ((IDX_SCALE - 1.0) - rem).astype(jnp.int32)


def _best_index(utts_q, utts_t, interpret=False):
    return pl.pallas_call(
        _argmax_body,
        grid=(NSUP // CHUNK,),
        in_specs=[
            pl.BlockSpec((NQ, SEQ), lambda i: (_i32(0), _i32(0))),
            pl.BlockSpec((CHUNK, FEAT), lambda i: (_i32(i), _i32(0))),
        ],
        out_specs=pl.BlockSpec((NQ, 1), lambda i: (_i32(0), _i32(0))),
        out_shape=jax.ShapeDtypeStruct((NQ, 1), jnp.int32),
        scratch_shapes=[
            pltpu.VMEM((NQ, FEAT), jnp.bfloat16),
            pltpu.VMEM((NQ, 1), jnp.float32),
        ],
        interpret=interpret,
    )(utts_q, utts_t)


_SC_CORES = 2
_SC_WORKERS = 16 * _SC_CORES
_BW = NQ // _SC_WORKERS
_GW = _BW // 16 * NTYPES * 16


def _gather_onehot_body(
    idx_hbm, tabw_hbm, out_hbm, idx_v, w0_v, w1_v, r0_v, r1_v, out_v, sem
):
    wid = lax.axis_index("s") * _SC_CORES + lax.axis_index("c")
    base = wid * _BW
    pltpu.sync_copy(idx_hbm.at[pl.ds(base, _BW)], idx_v)

    lanes = lax.iota(jnp.int32, 16)
    for g in range(2):
        q16 = idx_v[pl.ds(g * 16, 16)]
        wv = w0_v if g == 0 else w1_v
        for k in range(NTYPES):
            plsc.store_scatter(wv, [lanes * NTYPES + k], q16 + k * NSUP)
    cp0 = pltpu.async_copy(tabw_hbm.at[w0_v], r0_v, sem)
    cp1 = pltpu.async_copy(tabw_hbm.at[w1_v], r1_v, sem)

    zeros16 = jnp.zeros((16,), jnp.float32)
    for j in range(_BW * OUTW // 16):
        out_v[pl.ds(j * 16, 16)] = zeros16
    cp0.wait()
    cp1.wait()

    ones16 = jnp.ones((16,), jnp.float32)
    for g in range(2):
        rv = r0_v if g == 0 else r1_v
        for k in range(NTYPES):
            m16 = plsc.load_gather(rv, [lanes * NTYPES + k])
            oidx = (g * 16 + lanes) * OUTW + k * NMEAN + m16
            plsc.store_scatter(out_v, [oidx], ones16)

    pltpu.sync_copy(out_v, out_hbm.at[pl.ds(base * OUTW, _BW * OUTW)])


def _make_gather_onehot():
    mesh = plsc.VectorSubcoreMesh(
        core_axis_name="c", subcore_axis_name="s", num_cores=_SC_CORES
    )
    return pl.kernel(
        _gather_onehot_body,
        mesh=mesh,
        compiler_params=pltpu.CompilerParams(
            needs_layout_passes=False,
            use_tc_tiling_on_sc=False,
            skip_device_barrier=True,
        ),
        out_type=jax.ShapeDtypeStruct((NQ * OUTW,), jnp.float32),
        scratch_types=[
            pltpu.VMEM((_BW,), jnp.int32),
            pltpu.VMEM((_GW,), jnp.int32),
            pltpu.VMEM((_GW,), jnp.int32),
            pltpu.VMEM((_GW,), jnp.int32),
            pltpu.VMEM((_GW,), jnp.int32),
            pltpu.VMEM((_BW * OUTW,), jnp.float32),
            pltpu.SemaphoreType.DMA,
        ],
    )


def kernel(utts, utts_t, meanings_t):
    utts_q = jnp.transpose(utts).astype(jnp.int32)
    tabw = jnp.transpose(meanings_t, (1, 0)).astype(jnp.int32).reshape(-1)
    best = _best_index(utts_q, utts_t)
    idx = best.reshape(NQ)
    flat = _make_gather_onehot()(idx, tabw)
    return flat.reshape(NQ, NTYPES, NMEAN)


# speedup vs baseline: 12.1152x; 1.3188x over previous
import functools

import jax
import jax.numpy as jnp
from jax import lax
from jax.experimental import pallas as pl
from jax.experimental.pallas import tpu as pltpu
from jax.experimental.pallas import tpu_sc as plsc

VOCAB = 64
SEQ = 8
NQ = 1024
NSUP = 100000
FEAT = SEQ * VOCAB
CHUNK = 5000
NTYPES = 5
NMEAN = 10
OUTW = NTYPES * NMEAN
IDX_SCALE = 131072.0

def _i32(x):
    return jnp.asarray(x, jnp.int32)


def _argmax_body(utts_ref, sup_ref, out_ref, q1h_ref, best_ref):
    i = pl.program_id(0)
    nsteps = pl.num_programs(0)

    @pl.when(i == 0)
    def _init():
        best_ref[...] = jnp.full((NQ, 1), -1.0, jnp.float32)
        for p in range(SEQ):
            tok = utts_ref[:, p : p + 1]
            v = lax.broadcasted_iota(jnp.int32, (NQ, VOCAB), 1)
            q1h_ref[:, p * VOCAB : (p + 1) * VOCAB] = (
                (tok == v).astype(jnp.float32) * IDX_SCALE
            ).astype(jnp.bfloat16)

    chunk = sup_ref[...].astype(jnp.bfloat16)
    matches = lax.dot_general(
        q1h_ref[...],
        chunk,
        (((1,), (1,)), ((), ())),
        preferred_element_type=jnp.float32,
    )
    col = lax.broadcasted_iota(jnp.int32, (1, CHUNK), 1).astype(jnp.float32)
    base = (i * CHUNK).astype(jnp.float32)
    key = matches + ((IDX_SCALE - 1.0) - (base + col))
    step_best = jnp.max(key, axis=1, keepdims=True)
    best_ref[...] = jnp.maximum(best_ref[...], step_best)

    @pl.when(i == nsteps - 1)
    def _fin():
        b = best_ref[...]
        m = jnp.floor(b / IDX_SCALE)
        rem = b - m * IDX_SCALE
        out_ref[...] = ((IDX_SCALE - 1.0) - rem).astype(jnp.int32)


def _best_index(utts_q, utts_t, interpret=False):
    return pl.pallas_call(
        _argmax_body,
        grid=(NSUP // CHUNK,),
        in_specs=[
            pl.BlockSpec((NQ, SEQ), lambda i: (_i32(0), _i32(0))),
            pl.BlockSpec((CHUNK, FEAT), lambda i: (_i32(i), _i32(0))),
        ],
        out_specs=pl.BlockSpec((NQ, 1), lambda i: (_i32(0), _i32(0))),
        out_shape=jax.ShapeDtypeStruct((NQ, 1), jnp.int32),
        scratch_shapes=[
            pltpu.VMEM((NQ, FEAT), jnp.bfloat16),
            pltpu.VMEM((NQ, 1), jnp.float32),
        ],
        interpret=interpret,
    )(utts_q, utts_t)


_SC_CORES = 2
_SC_WORKERS = 16 * _SC_CORES
_BW = NQ // _SC_WORKERS
_GW = _BW // 16 * NTYPES * 16


def _gather_onehot_body(idx_hbm, code_hbm, out_hbm, idx_v, c_v, out_v, sem):
    wid = lax.axis_index("s") * _SC_CORES + lax.axis_index("c")
    base = wid * _BW
    pltpu.sync_copy(idx_hbm.at[pl.ds(base, _BW)], idx_v)
    cp = pltpu.async_copy(code_hbm.at[idx_v], c_v, sem)

    zeros16 = jnp.zeros((16,), jnp.float32)
    for j in range(_BW * OUTW // 16):
        out_v[pl.ds(j * 16, 16)] = zeros16
    cp.wait()

    ones16 = jnp.ones((16,), jnp.float32)
    lanes = lax.iota(jnp.int32, 16)
    for g in range(2):
        c16 = c_v[pl.ds(g * 16, 16)]
        for k in range(NTYPES):
            m16 = lax.shift_right_logical(c16, jnp.int32(4 * k)) & jnp.int32(15)
            oidx = (g * 16 + lanes) * OUTW + k * NMEAN + m16
            plsc.store_scatter(out_v, [oidx], ones16)

    pltpu.sync_copy(out_v, out_hbm.at[pl.ds(base * OUTW, _BW * OUTW)])


def _make_gather_onehot():
    mesh = plsc.VectorSubcoreMesh(
        core_axis_name="c", subcore_axis_name="s", num_cores=_SC_CORES
    )
    return pl.kernel(
        _gather_onehot_body,
        mesh=mesh,
        compiler_params=pltpu.CompilerParams(
            needs_layout_passes=False,
            use_tc_tiling_on_sc=False,
            skip_device_barrier=True,
        ),
        out_type=jax.ShapeDtypeStruct((NQ * OUTW,), jnp.float32),
        scratch_types=[
            pltpu.VMEM((_BW,), jnp.int32),
            pltpu.VMEM((_BW,), jnp.int32),
            pltpu.VMEM((_BW * OUTW,), jnp.float32),
            pltpu.SemaphoreType.DMA,
        ],
    )


def kernel(utts, utts_t, meanings_t):
    utts_q = jnp.transpose(utts).astype(jnp.int32)
    m32 = jnp.transpose(meanings_t, (1, 0)).astype(jnp.int32)
    nib = jnp.array([1, 16, 256, 4096, 65536], jnp.int32)
    code = jnp.sum(m32 * nib[:, None], axis=0, dtype=jnp.int32)
    best = _best_index(utts_q, utts_t)
    idx = best.reshape(NQ)
    flat = _make_gather_onehot()(idx, code)
    return flat.reshape(NQ, NTYPES, NMEAN)


# speedup vs baseline: 12.1468x; 1.0026x over previous
import functools

import jax
import jax.numpy as jnp
from jax import lax
from jax.experimental import pallas as pl
from jax.experimental.pallas import tpu as pltpu
from jax.experimental.pallas import tpu_sc as plsc

VOCAB = 64
SEQ = 8
NQ = 1024
NSUP = 100000
FEAT = SEQ * VOCAB
CHUNK = 5000
NTYPES = 5
NMEAN = 10
OUTW = NTYPES * NMEAN
IDX_SCALE = 131072.0

def _i32(x):
    return jnp.asarray(x, jnp.int32)


def _argmax_body(utts_ref, sup_ref, out_ref, q1h_ref, best_ref):
    i = pl.program_id(0)
    nsteps = pl.num_programs(0)

    @pl.when(i == 0)
    def _init():
        best_ref[...] = jnp.full((NQ, 1), -1.0, jnp.float32)
        for p in range(SEQ):
            tok = utts_ref[:, p : p + 1]
            v = lax.broadcasted_iota(jnp.int32, (NQ, VOCAB), 1)
            q1h_ref[:, p * VOCAB : (p + 1) * VOCAB] = (
                (tok == v).astype(jnp.float32) * IDX_SCALE
            )

    matches = lax.dot_general(
        q1h_ref[...],
        sup_ref[...],
        (((1,), (1,)), ((), ())),
        precision=lax.Precision.DEFAULT,
        preferred_element_type=jnp.float32,
    )
    col = lax.broadcasted_iota(jnp.int32, (1, CHUNK), 1).astype(jnp.float32)
    base = (i * CHUNK).astype(jnp.float32)
    key = matches + ((IDX_SCALE - 1.0) - (base + col))
    step_best = jnp.max(key, axis=1, keepdims=True)
    best_ref[...] = jnp.maximum(best_ref[...], step_best)

    @pl.when(i == nsteps - 1)
    def _fin():
        b = best_ref[...]
        m = jnp.floor(b / IDX_SCALE)
        rem = b - m * IDX_SCALE
        out_ref[...] = ((IDX_SCALE - 1.0) - rem).astype(jnp.int32)


def _best_index(utts_q, utts_t, interpret=False):
    return pl.pallas_call(
        _argmax_body,
        grid=(NSUP // CHUNK,),
        in_specs=[
            pl.BlockSpec((NQ, SEQ), lambda i: (_i32(0), _i32(0))),
            pl.BlockSpec((CHUNK, FEAT), lambda i: (_i32(i), _i32(0))),
        ],
        out_specs=pl.BlockSpec((NQ, 1), lambda i: (_i32(0), _i32(0))),
        out_shape=jax.ShapeDtypeStruct((NQ, 1), jnp.int32),
        scratch_shapes=[
            pltpu.VMEM((NQ, FEAT), jnp.float32),
            pltpu.VMEM((NQ, 1), jnp.float32),
        ],
        interpret=interpret,
    )(utts_q, utts_t)


_SC_CORES = 2
_SC_WORKERS = 16 * _SC_CORES
_BW = NQ // _SC_WORKERS
_GW = _BW // 16 * NTYPES * 16


def _gather_onehot_body(idx_hbm, code_hbm, out_hbm, idx_v, c_v, out_v, sem):
    wid = lax.axis_index("s") * _SC_CORES + lax.axis_index("c")
    base = wid * _BW
    pltpu.sync_copy(idx_hbm.at[pl.ds(base, _BW)], idx_v)
    cp = pltpu.async_copy(code_hbm.at[idx_v], c_v, sem)

    zeros16 = jnp.zeros((16,), jnp.float32)
    for j in range(_BW * OUTW // 16):
        out_v[pl.ds(j * 16, 16)] = zeros16
    cp.wait()

    ones16 = jnp.ones((16,), jnp.float32)
    lanes = lax.iota(jnp.int32, 16)
    for g in range(2):
        c16 = c_v[pl.ds(g * 16, 16)]
        for k in range(NTYPES):
            m16 = lax.shift_right_logical(c16, jnp.int32(4 * k)) & jnp.int32(15)
            oidx = (g * 16 + lanes) * OUTW + k * NMEAN + m16
            plsc.store_scatter(out_v, [oidx], ones16)

    pltpu.sync_copy(out_v, out_hbm.at[pl.ds(base * OUTW, _BW * OUTW)])


def _make_gather_onehot():
    mesh = plsc.VectorSubcoreMesh(
        core_axis_name="c", subcore_axis_name="s", num_cores=_SC_CORES
    )
    return pl.kernel(
        _gather_onehot_body,
        mesh=mesh,
        compiler_params=pltpu.CompilerParams(
            needs_layout_passes=False,
            use_tc_tiling_on_sc=False,
            skip_device_barrier=True,
        ),
        out_type=jax.ShapeDtypeStruct((NQ * OUTW,), jnp.float32),
        scratch_types=[
            pltpu.VMEM((_BW,), jnp.int32),
            pltpu.VMEM((_BW,), jnp.int32),
            pltpu.VMEM((_BW * OUTW,), jnp.float32),
            pltpu.SemaphoreType.DMA,
        ],
    )


def kernel(utts, utts_t, meanings_t):
    utts_q = jnp.transpose(utts).astype(jnp.int32)
    m32 = jnp.transpose(meanings_t, (1, 0)).astype(jnp.int32)
    nib = jnp.array([1, 16, 256, 4096, 65536], jnp.int32)
    code = jnp.sum(m32 * nib[:, None], axis=0, dtype=jnp.int32)
    best = _best_index(utts_q, utts_t)
    idx = best.reshape(NQ)
    flat = _make_gather_onehot()(idx, code)
    return flat.reshape(NQ, NTYPES, NMEAN)


# speedup vs baseline: 12.1522x; 1.0004x over previous
import jax
import jax.numpy as jnp
from jax import lax
from jax.experimental import pallas as pl
from jax.experimental.pallas import tpu as pltpu
from jax.experimental.pallas import tpu_sc as plsc

VOCAB = 64
SEQ = 8
NQ = 1024
NSUP = 100000
FEAT = SEQ * VOCAB
CHUNK = 5000
NTYPES = 5
NMEAN = 10
OUTW = NTYPES * NMEAN
IDX_SCALE = 131072.0

def _i32(x):
    return jnp.asarray(x, jnp.int32)


def _argmax_body(utts_ref, sup_ref, out_ref, q1h_ref, best_ref):
    i = pl.program_id(0)
    nsteps = pl.num_programs(0)

    @pl.when(i == 0)
    def _init():
        best_ref[...] = jnp.full((NQ, 1), -1.0, jnp.float32)
        for p in range(SEQ):
            tok = utts_ref[:, p : p + 1]
            v = lax.broadcasted_iota(jnp.int32, (NQ, VOCAB), 1)
            q1h_ref[:, p * VOCAB : (p + 1) * VOCAB] = (
                (tok == v).astype(jnp.float32) * IDX_SCALE
            )

    matches = lax.dot_general(
        q1h_ref[...],
        sup_ref[...],
        (((1,), (1,)), ((), ())),
        precision=lax.Precision.DEFAULT,
        preferred_element_type=jnp.float32,
    )
    col = lax.broadcasted_iota(jnp.int32, (1, CHUNK), 1).astype(jnp.float32)
    base = (i * CHUNK).astype(jnp.float32)
    key = matches + ((IDX_SCALE - 1.0) - (base + col))
    step_best = jnp.max(key, axis=1, keepdims=True)
    best_ref[...] = jnp.maximum(best_ref[...], step_best)

    @pl.when(i == nsteps - 1)
    def _fin():
        b = best_ref[...]
        m = jnp.floor(b / IDX_SCALE)
        rem = b - m * IDX_SCALE
        out_ref[...] = ((IDX_SCALE - 1.0) - rem).astype(jnp.int32)


def _best_index(utts_q, utts_t, interpret=False):
    return pl.pallas_call(
        _argmax_body,
        grid=(NSUP // CHUNK,),
        in_specs=[
            pl.BlockSpec((NQ, SEQ), lambda i: (_i32(0), _i32(0))),
            pl.BlockSpec((CHUNK, FEAT), lambda i: (_i32(i), _i32(0))),
        ],
        out_specs=pl.BlockSpec((NQ, 1), lambda i: (_i32(0), _i32(0))),
        out_shape=jax.ShapeDtypeStruct((NQ, 1), jnp.int32),
        scratch_shapes=[
            pltpu.VMEM((NQ, FEAT), jnp.float32),
            pltpu.VMEM((NQ, 1), jnp.float32),
        ],
        interpret=interpret,
    )(utts_q, utts_t)


_SC_CORES = 2
_SC_WORKERS = 16 * _SC_CORES
_BW = NQ // _SC_WORKERS


def _gather_onehot_body(idx_hbm, code_hbm, out_hbm, idx_v, c_v, out_v, sem):
    wid = lax.axis_index("s") * _SC_CORES + lax.axis_index("c")
    base = wid * _BW
    pltpu.sync_copy(idx_hbm.at[pl.ds(base, _BW)], idx_v)
    cp = pltpu.async_copy(code_hbm.at[idx_v], c_v, sem)

    zeros16 = jnp.zeros((16,), jnp.float32)
    for j in range(_BW * OUTW // 16):
        out_v[pl.ds(j * 16, 16)] = zeros16
    cp.wait()

    ones16 = jnp.ones((16,), jnp.float32)
    lanes = lax.iota(jnp.int32, 16)
    for g in range(2):
        c16 = c_v[pl.ds(g * 16, 16)]
        for k in range(NTYPES):
            m16 = lax.shift_right_logical(c16, jnp.int32(4 * k)) & jnp.int32(15)
            oidx = (g * 16 + lanes) * OUTW + k * NMEAN + m16
            plsc.store_scatter(out_v, [oidx], ones16)

    pltpu.sync_copy(out_v, out_hbm.at[pl.ds(base * OUTW, _BW * OUTW)])


def _make_gather_onehot():
    mesh = plsc.VectorSubcoreMesh(
        core_axis_name="c", subcore_axis_name="s", num_cores=_SC_CORES
    )
    return pl.kernel(
        _gather_onehot_body,
        mesh=mesh,
        compiler_params=pltpu.CompilerParams(
            needs_layout_passes=False,
            use_tc_tiling_on_sc=False,
            skip_device_barrier=True,
        ),
        out_type=jax.ShapeDtypeStruct((NQ * OUTW,), jnp.float32),
        scratch_types=[
            pltpu.VMEM((_BW,), jnp.int32),
            pltpu.VMEM((_BW,), jnp.int32),
            pltpu.VMEM((_BW * OUTW,), jnp.float32),
            pltpu.SemaphoreType.DMA,
        ],
    )


def kernel(utts, utts_t, meanings_t):
    utts_q = jnp.transpose(utts).astype(jnp.int32)
    m32 = jnp.transpose(meanings_t, (1, 0)).astype(jnp.int32)
    nib = jnp.array([1, 16, 256, 4096, 65536], jnp.int32)
    code = jnp.sum(m32 * nib[:, None], axis=0, dtype=jnp.int32)
    best = _best_index(utts_q, utts_t)
    idx = best.reshape(NQ)
    flat = _make_gather_onehot()(idx, code)
    return flat.reshape(NQ, NTYPES, NMEAN)
